# baseline (speedup 1.0000x reference)
import functools

import jax
import jax.numpy as jnp
from jax import lax
from jax.experimental import pallas as pl
from jax.experimental.pallas import tpu as pltpu
from jax.experimental.pallas import tpu_sc as plsc

VOCAB = 1000000
EMBED_DIM = 32
BATCH = 16384
HIST_LEN = 50

NC = 2
NS = 16
NW = NC * NS
B_PER_W = BATCH // NW
NBLK = B_PER_W // 128
NGF = EMBED_DIM // 8
NPAIRS = HIST_LEN // 2

_mesh = plsc.VectorSubcoreMesh(core_axis_name="c", subcore_axis_name="s")


@functools.partial(
    pl.kernel,
    out_type=jax.ShapeDtypeStruct((HIST_LEN, NGF, NW, NBLK * 8 * 128),
                                  jnp.float32),
    mesh=_mesh,
    scratch_types=[
        pltpu.VMEM((B_PER_W, HIST_LEN), jnp.int32),
        pltpu.VMEM((HIST_LEN * B_PER_W,), jnp.int32),
        pltpu.VMEM((B_PER_W, EMBED_DIM), jnp.float32),
        pltpu.VMEM((B_PER_W, EMBED_DIM), jnp.float32),
        pltpu.VMEM((NGF, NBLK * 8 * 128), jnp.float32),
        pltpu.VMEM((NGF, NBLK * 8 * 128), jnp.float32),
        pltpu.SemaphoreType.DMA,
        pltpu.SemaphoreType.DMA,
        pltpu.SemaphoreType.DMA,
        pltpu.SemaphoreType.DMA,
        pltpu.SemaphoreType.DMA,
    ],
    compiler_params=pltpu.CompilerParams(use_tc_tiling_on_sc=False,
                                         needs_layout_passes=False),
)
def _emb_gather(title_hbm, table_hbm, out_hbm, idx_b, idx_t, rows0, rows1,
                tiles0, tiles1, isem, gsem0, gsem1, wsem0, wsem1):
    wid = lax.axis_index("s") * NC + lax.axis_index("c")
    base = pl.multiple_of(wid * B_PER_W, B_PER_W)

    lanes = lax.iota(jnp.int32, 16)
    pos16 = (lanes % 8) * 128
    g_lo = lanes // 8
    g_hi = g_lo + 2

    pltpu.async_copy(title_hbm.at[pl.ds(base, B_PER_W)], idx_b, isem).wait()

    def tr_idx_body(t, carry):
        tb = t * B_PER_W
        col = lanes * 0 + t
        for m in range(B_PER_W // 16):
            src = plsc.load_gather(idx_b, [lanes + 16 * m, col])
            idx_t[pl.ds(tb + 16 * m, 16)] = src
        return carry

    lax.fori_loop(0, HIST_LEN, tr_idx_body, 0)

    rows = (rows0, rows1)
    tiles = (tiles0, tiles1)
    gsem = (gsem0, gsem1)
    wsem = (wsem0, wsem1)

    def fire_gathers(t, slot):
        for k in range(NBLK):
            pltpu.async_copy(
                table_hbm.at[idx_t.at[pl.ds(t * B_PER_W + k * 128, 128)]],
                rows[slot].at[pl.ds(k * 128, 128)],
                gsem[slot],
            )

    def drain_gathers(slot):
        for k in range(NBLK):
            pltpu.make_async_copy(
                table_hbm.at[pl.ds(0, 128)],
                rows[slot].at[pl.ds(k * 128, 128)],
                gsem[slot],
            ).wait()

    def transpose_rows(slot):
        rv = rows[slot]
        tv = tiles[slot]

        def body(i, carry):
            for u in range(8):
                b = i * 8 + u
                pos = pos16 + ((b // 128) * 1024 + (b % 128))
                v_lo = rv[b, pl.ds(0, 16)]
                v_hi = rv[b, pl.ds(16, 16)]
                plsc.store_scatter(tv, [g_lo, pos], v_lo)
                plsc.store_scatter(tv, [g_hi, pos], v_hi)
            return carry

        lax.fori_loop(0, B_PER_W // 8, body, 0)

    def fire_write(t, slot):
        for g in range(NGF):
            pltpu.async_copy(
                tiles[slot].at[g], out_hbm.at[t, g, wid], wsem[slot]
            )

    def drain_write(slot):
        for g in range(NGF):
            pltpu.make_async_copy(
                tiles[slot].at[g], out_hbm.at[0, 0, 0], wsem[slot]
            ).wait()

    fire_gathers(0, 0)

    def pair_body(p, carry):
        t0 = p * 2
        t1 = t0 + 1

        @pl.when(p > 0)
        def _():
            drain_write(1)

        fire_gathers(t1, 1)

        drain_gathers(0)
        transpose_rows(0)
        fire_write(t0, 0)

        @pl.when(p + 1 < NPAIRS)
        def _():
            drain_write(0)
            fire_gathers(t0 + 2, 0)

        drain_gathers(1)
        transpose_rows(1)
        fire_write(t1, 1)
        return carry

    lax.fori_loop(0, NPAIRS, pair_body, 0)
    drain_write(0)
    drain_write(1)


def kernel(title, table):
    x = _emb_gather(title.astype(jnp.int32), table)
    x = x.reshape(HIST_LEN, NGF, BATCH // 128, 8, 128)
    return x.transpose(2, 4, 0, 1, 3).reshape(BATCH, HIST_LEN, EMBED_DIM)


# speedup vs baseline: 1.0835x; 1.0835x over previous
import functools

import jax
import jax.numpy as jnp
from jax import lax
from jax.experimental import pallas as pl
from jax.experimental.pallas import tpu as pltpu
from jax.experimental.pallas import tpu_sc as plsc

VOCAB = 1000000
EMBED_DIM = 32
BATCH = 16384
HIST_LEN = 50

NC = 2
NS = 16
NW = NC * NS
B_PER_W = BATCH // NW
NBLK = B_PER_W // 128
NGF = EMBED_DIM // 8
NPAIRS = HIST_LEN // 2

_mesh = plsc.VectorSubcoreMesh(core_axis_name="c", subcore_axis_name="s")


@functools.partial(
    pl.kernel,
    out_type=jax.ShapeDtypeStruct((HIST_LEN, NGF, NW, NBLK * 8 * 128),
                                  jnp.float32),
    mesh=_mesh,
    scratch_types=[
        pltpu.VMEM((B_PER_W, HIST_LEN), jnp.int32),
        pltpu.VMEM((HIST_LEN * B_PER_W,), jnp.int32),
        pltpu.VMEM((B_PER_W, EMBED_DIM), jnp.float32),
        pltpu.VMEM((B_PER_W, EMBED_DIM), jnp.float32),
        pltpu.VMEM((NGF, NBLK * 8 * 128), jnp.float32),
        pltpu.VMEM((NGF, NBLK * 8 * 128), jnp.float32),
        pltpu.SemaphoreType.DMA,
        pltpu.SemaphoreType.DMA,
        pltpu.SemaphoreType.DMA,
        pltpu.SemaphoreType.DMA,
        pltpu.SemaphoreType.DMA,
    ],
    compiler_params=pltpu.CompilerParams(use_tc_tiling_on_sc=False,
                                         needs_layout_passes=False),
)
def _emb_gather(title_hbm, table_hbm, out_hbm, idx_b, idx_t, rows0, rows1,
                tiles0, tiles1, isem, gsem0, gsem1, wsem0, wsem1):
    wid = lax.axis_index("s") * NC + lax.axis_index("c")
    base = pl.multiple_of(wid * B_PER_W, B_PER_W)

    lanes = lax.iota(jnp.int32, 16)
    pos16 = (lanes % 8) * 128
    g_lo = lanes // 8
    g_hi = g_lo + 2

    pltpu.async_copy(title_hbm.at[pl.ds(base, B_PER_W)], idx_b, isem).wait()

    @plsc.parallel_loop(0, HIST_LEN, unroll=2)
    def _(t):
        tb = t * B_PER_W
        col = lanes * 0 + t
        for m in range(B_PER_W // 16):
            src = plsc.load_gather(idx_b, [lanes + 16 * m, col])
            idx_t[pl.ds(tb + 16 * m, 16)] = src

    rows = (rows0, rows1)
    tiles = (tiles0, tiles1)
    gsem = (gsem0, gsem1)
    wsem = (wsem0, wsem1)

    def fire_gathers(t, slot):
        for k in range(NBLK):
            pltpu.async_copy(
                table_hbm.at[idx_t.at[pl.ds(t * B_PER_W + k * 128, 128)]],
                rows[slot].at[pl.ds(k * 128, 128)],
                gsem[slot],
            )

    def drain_gathers(slot):
        for k in range(NBLK):
            pltpu.make_async_copy(
                table_hbm.at[pl.ds(0, 128)],
                rows[slot].at[pl.ds(k * 128, 128)],
                gsem[slot],
            ).wait()

    def transpose_rows(slot):
        rv = rows[slot]
        tv = tiles[slot]

        @plsc.parallel_loop(0, B_PER_W, unroll=8)
        def _(b):
            pos = pos16 + ((b // 128) * 1024 + (b % 128))
            v_lo = rv[b, pl.ds(0, 16)]
            v_hi = rv[b, pl.ds(16, 16)]
            plsc.store_scatter(tv, [g_lo, pos], v_lo)
            plsc.store_scatter(tv, [g_hi, pos], v_hi)

    def fire_write(t, slot):
        for g in range(NGF):
            pltpu.async_copy(
                tiles[slot].at[g], out_hbm.at[t, g, wid], wsem[slot]
            )

    def drain_write(slot):
        for g in range(NGF):
            pltpu.make_async_copy(
                tiles[slot].at[g], out_hbm.at[0, 0, 0], wsem[slot]
            ).wait()

    fire_gathers(0, 0)

    def pair_body(p, carry):
        t0 = p * 2
        t1 = t0 + 1

        @pl.when(p > 0)
        def _():
            drain_write(1)

        fire_gathers(t1, 1)

        drain_gathers(0)
        transpose_rows(0)
        fire_write(t0, 0)

        @pl.when(p + 1 < NPAIRS)
        def _():
            drain_write(0)
            fire_gathers(t0 + 2, 0)

        drain_gathers(1)
        transpose_rows(1)
        fire_write(t1, 1)
        return carry

    lax.fori_loop(0, NPAIRS, pair_body, 0)
    drain_write(0)
    drain_write(1)


def kernel(title, table):
    x = _emb_gather(title.astype(jnp.int32), table)
    x = x.reshape(HIST_LEN, NGF, BATCH // 128, 8, 128)
    return x.transpose(2, 4, 0, 1, 3).reshape(BATCH, HIST_LEN, EMBED_DIM)


# speedup vs baseline: 1.1518x; 1.0630x over previous
import functools

import jax
import jax.numpy as jnp
from jax import lax
from jax.experimental import pallas as pl
from jax.experimental.pallas import tpu as pltpu
from jax.experimental.pallas import tpu_sc as plsc

VOCAB = 1000000
EMBED_DIM = 32
BATCH = 16384
HIST_LEN = 50

NC = 2
NS = 16
NW = NC * NS
B_PER_W = BATCH // NW
NBLK = B_PER_W // 128
NGF = EMBED_DIM // 8
NPAIRS = HIST_LEN // 2

_mesh = plsc.VectorSubcoreMesh(core_axis_name="c", subcore_axis_name="s")


@functools.partial(
    pl.kernel,
    out_type=jax.ShapeDtypeStruct((HIST_LEN, NGF, NW, NBLK * 8 * 128),
                                  jnp.float32),
    mesh=_mesh,
    scratch_types=[
        pltpu.VMEM((B_PER_W, HIST_LEN), jnp.int32),
        pltpu.VMEM((HIST_LEN * B_PER_W,), jnp.int32),
        pltpu.VMEM((B_PER_W, EMBED_DIM), jnp.float32),
        pltpu.VMEM((B_PER_W, EMBED_DIM), jnp.float32),
        pltpu.VMEM((NGF * NBLK * 8 * 128,), jnp.float32),
        pltpu.VMEM((NGF * NBLK * 8 * 128,), jnp.float32),
        pltpu.SemaphoreType.DMA,
        pltpu.SemaphoreType.DMA,
        pltpu.SemaphoreType.DMA,
        pltpu.SemaphoreType.DMA,
        pltpu.SemaphoreType.DMA,
    ],
    compiler_params=pltpu.CompilerParams(use_tc_tiling_on_sc=False,
                                         needs_layout_passes=False),
)
def _emb_gather(title_hbm, table_hbm, out_hbm, idx_b, idx_t, rows0, rows1,
                tiles0, tiles1, isem, gsem0, gsem1, wsem0, wsem1):
    wid = lax.axis_index("s") * NC + lax.axis_index("c")
    base = pl.multiple_of(wid * B_PER_W, B_PER_W)

    lanes = lax.iota(jnp.int32, 16)
    pos_lo = (lanes // 8) * 4096 + (lanes % 8) * 128
    pos_hi = pos_lo + 2 * 4096

    pltpu.async_copy(title_hbm.at[pl.ds(base, B_PER_W)], idx_b, isem).wait()

    @plsc.parallel_loop(0, HIST_LEN, unroll=2)
    def _(t):
        tb = t * B_PER_W
        col = lanes * 0 + t
        for m in range(B_PER_W // 16):
            src = plsc.load_gather(idx_b, [lanes + 16 * m, col])
            idx_t[pl.ds(tb + 16 * m, 16)] = src

    rows = (rows0, rows1)
    tiles = (tiles0, tiles1)
    gsem = (gsem0, gsem1)
    wsem = (wsem0, wsem1)

    def fire_gathers(t, slot):
        for k in range(NBLK):
            pltpu.async_copy(
                table_hbm.at[idx_t.at[pl.ds(t * B_PER_W + k * 128, 128)]],
                rows[slot].at[pl.ds(k * 128, 128)],
                gsem[slot],
            )

    def drain_gathers(slot):
        for k in range(NBLK):
            pltpu.make_async_copy(
                table_hbm.at[pl.ds(0, 128)],
                rows[slot].at[pl.ds(k * 128, 128)],
                gsem[slot],
            ).wait()

    def transpose_rows(slot):
        rv = rows[slot]
        tv = tiles[slot]
        for blk in range(NBLK):
            base_lo = pos_lo + blk * 1024
            base_hi = pos_hi + blk * 1024

            @plsc.parallel_loop(0, 128, unroll=8)
            def _(j, blk=blk, base_lo=base_lo, base_hi=base_hi):
                b = blk * 128 + j
                plsc.store_scatter(tv, [base_lo + j], rv[b, pl.ds(0, 16)])
                plsc.store_scatter(tv, [base_hi + j], rv[b, pl.ds(16, 16)])

    def fire_write(t, slot):
        for g in range(NGF):
            pltpu.async_copy(
                tiles[slot].at[pl.ds(g * 4096, 4096)],
                out_hbm.at[t, g, wid],
                wsem[slot],
            )

    def drain_write(slot):
        for g in range(NGF):
            pltpu.make_async_copy(
                tiles[slot].at[pl.ds(g * 4096, 4096)],
                out_hbm.at[0, 0, 0],
                wsem[slot],
            ).wait()

    fire_gathers(0, 0)

    def pair_body(p, carry):
        t0 = p * 2
        t1 = t0 + 1

        @pl.when(p > 0)
        def _():
            drain_write(1)

        fire_gathers(t1, 1)

        drain_gathers(0)
        transpose_rows(0)
        fire_write(t0, 0)

        @pl.when(p + 1 < NPAIRS)
        def _():
            drain_write(0)
            fire_gathers(t0 + 2, 0)

        drain_gathers(1)
        transpose_rows(1)
        fire_write(t1, 1)
        return carry

    lax.fori_loop(0, NPAIRS, pair_body, 0)
    drain_write(0)
    drain_write(1)


def kernel(title, table):
    x = _emb_gather(title.astype(jnp.int32), table)
    x = x.reshape(HIST_LEN, NGF, BATCH // 128, 8, 128)
    return x.transpose(2, 4, 0, 1, 3).reshape(BATCH, HIST_LEN, EMBED_DIM)
